# baseline (speedup 1.0000x reference)
import functools

import jax
import jax.numpy as jnp
from jax import lax
from jax.experimental import pallas as pl
from jax.experimental.pallas import tpu as pltpu
from jax.experimental.pallas import tpu_sc as plsc

EMB_DIM = 64
BATCH = 4096
SEQ = 200
SEQ_PAD = 208

_info = plsc.get_sparse_core_info()
NC, NS, NL = _info.num_cores, _info.num_subcores, _info.num_lanes
NW = NC * NS
BPW = BATCH // NW


HSEQ = SEQ_PAD // 2


def _sc_pool_body(ids_hbm, emb_hbm, sums_hbm, idx0_v, idx1_v,
                  rows0_v, rows1_v, sums_v, sem0, sem1):
    wid = lax.axis_index("s") * NC + lax.axis_index("c")
    base = wid * BPW

    def batch_body(b, carry):
        off = pl.multiple_of((base + b) * SEQ_PAD, 8)
        pltpu.sync_copy(ids_hbm.at[pl.ds(off, HSEQ)], idx0_v)
        pltpu.sync_copy(ids_hbm.at[pl.ds(off + HSEQ, HSEQ)], idx1_v)
        c0 = pltpu.async_copy(emb_hbm.at[idx0_v], rows0_v, sem0)
        c1 = pltpu.async_copy(emb_hbm.at[idx1_v], rows1_v, sem1)
        c0.wait()
        c1.wait()

        def row_body(r, accs):
            a0, a1, a2, a3 = accs
            return (a0 + rows0_v[r, pl.ds(0, 16)] + rows1_v[r, pl.ds(0, 16)],
                    a1 + rows0_v[r, pl.ds(16, 16)] + rows1_v[r, pl.ds(16, 16)],
                    a2 + rows0_v[r, pl.ds(32, 16)] + rows1_v[r, pl.ds(32, 16)],
                    a3 + rows0_v[r, pl.ds(48, 16)] + rows1_v[r, pl.ds(48, 16)])

        z = jnp.zeros((16,), jnp.float32)
        a0, a1, a2, a3 = lax.fori_loop(0, HSEQ, row_body, (z, z, z, z),
                                       unroll=4)
        sums_v[b, pl.ds(0, 16)] = a0
        sums_v[b, pl.ds(16, 16)] = a1
        sums_v[b, pl.ds(32, 16)] = a2
        sums_v[b, pl.ds(48, 16)] = a3
        return carry

    lax.fori_loop(0, BPW, batch_body, 0)
    pltpu.sync_copy(sums_v, sums_hbm.at[pl.ds(base, BPW)])


_sc_pool = functools.partial(
    pl.kernel,
    out_type=jax.ShapeDtypeStruct((BATCH, EMB_DIM), jnp.float32),
    mesh=plsc.VectorSubcoreMesh(core_axis_name="c", subcore_axis_name="s"),
    compiler_params=pltpu.CompilerParams(use_tc_tiling_on_sc=False),
    scratch_types=[
        pltpu.VMEM((HSEQ,), jnp.int32),
        pltpu.VMEM((HSEQ,), jnp.int32),
        pltpu.VMEM((HSEQ, EMB_DIM), jnp.float32),
        pltpu.VMEM((HSEQ, EMB_DIM), jnp.float32),
        pltpu.VMEM((BPW, EMB_DIM), jnp.float32),
        pltpu.SemaphoreType.DMA,
        pltpu.SemaphoreType.DMA,
    ],
)(_sc_pool_body)


def _mlp_body(ids_ref, sums_ref, w1_ref, b1_ref, w2_ref, b2_ref, out_ref):
    cnt = jnp.sum((ids_ref[...] != 0).astype(jnp.float32), axis=1,
                  keepdims=True)
    avg = sums_ref[...] / jnp.maximum(cnt, 1e-6)
    h = jnp.dot(avg, w1_ref[...], preferred_element_type=jnp.float32,
                precision=lax.Precision.HIGHEST) + b1_ref[...]
    h = jnp.maximum(h, 0.0)
    out_ref[...] = jnp.dot(h, w2_ref[...], preferred_element_type=jnp.float32,
                           precision=lax.Precision.HIGHEST) + b2_ref[...]


def kernel(ids, emb, W1, b1, W2, b2):
    ids = ids.astype(jnp.int32)
    idsp = jnp.pad(ids, ((0, 0), (0, SEQ_PAD - SEQ)))
    sums = _sc_pool(idsp.reshape(-1), emb)

    blk = 512
    grid = (BATCH // blk,)
    hidden = W1.shape[1]
    out_dim = W2.shape[1]
    out = pl.pallas_call(
        _mlp_body,
        grid=grid,
        in_specs=[
            pl.BlockSpec((blk, SEQ_PAD), lambda i: (i, 0)),
            pl.BlockSpec((blk, EMB_DIM), lambda i: (i, 0)),
            pl.BlockSpec((EMB_DIM, hidden), lambda i: (0, 0)),
            pl.BlockSpec((1, hidden), lambda i: (0, 0)),
            pl.BlockSpec((hidden, out_dim), lambda i: (0, 0)),
            pl.BlockSpec((1, out_dim), lambda i: (0, 0)),
        ],
        out_specs=pl.BlockSpec((blk, out_dim), lambda i: (i, 0)),
        out_shape=jax.ShapeDtypeStruct((BATCH, out_dim), jnp.float32),
    )(idsp, sums, W1, b1[None, :], W2, b2[None, :])
    return out
